# Initial kernel scaffold; baseline (speedup 1.0000x reference)
import jax
import jax.numpy as jnp
from jax.experimental import pallas as pl


def kernel(x, Wg, bg, Wn, bn, W1, b1, W2, b2, noise):
    raise NotImplementedError("write your pallas kernel here")


# speedup vs baseline: 2.4432x; 2.4432x over previous
import functools
import math

import jax
import jax.numpy as jnp
from jax.experimental import pallas as pl
from jax.experimental.pallas import tpu as pltpu

S, D, E, K, H = 2048, 1024, 8, 2, 4096
HC = 512
NH = H // HC

_HIGH = jax.lax.Precision.HIGHEST


def _softplus(z):
    return jnp.maximum(z, 0.0) + jnp.log1p(jnp.exp(-jnp.abs(z)))


def _router_body(noisy_ref, p_ref):
    noisy = noisy_ref[...]

    lanes = jax.lax.broadcasted_iota(jnp.int32, (S, E), 1)
    v0 = jnp.max(noisy, axis=1, keepdims=True)
    e0 = jnp.min(jnp.where(noisy == v0, lanes, E), axis=1, keepdims=True)
    oh0 = lanes == e0
    masked = jnp.where(oh0, -jnp.inf, noisy)
    v1 = jnp.max(masked, axis=1, keepdims=True)
    e1 = jnp.min(jnp.where(masked == v1, lanes, E), axis=1, keepdims=True)
    oh1 = lanes == e1

    z = jnp.exp(v1 - v0)
    p0 = 1.0 / (1.0 + z)
    p1 = z / (1.0 + z)
    p_ref[...] = jnp.where(oh0, p0, 0.0) + jnp.where(oh1, p1, 0.0)


def _gelu(v):
    return 0.5 * v * (1.0 + jax.lax.erf(v * (1.0 / math.sqrt(2.0))))


def _expert_body(x_ref, p_ref, w1_ref, b1_ref, w2_ref, b2_ref, out_ref):
    e = pl.program_id(0)
    j = pl.program_id(1)
    h = jnp.dot(x_ref[...], w1_ref[0], preferred_element_type=jnp.float32)
    h = _gelu(h + b1_ref[0])
    contrib = jnp.dot(h, w2_ref[0], preferred_element_type=jnp.float32)

    lanes = jax.lax.broadcasted_iota(jnp.int32, (S, E), 1)
    pe = jnp.sum(jnp.where(lanes == e, p_ref[...], 0.0), axis=1, keepdims=True)
    contrib = pe * (contrib + jnp.where(j == 0, 1.0, 0.0) * b2_ref[0])

    @pl.when((e == 0) & (j == 0))
    def _():
        out_ref[...] = contrib

    @pl.when((e > 0) | (j > 0))
    def _():
        out_ref[...] = out_ref[...] + contrib


def _run(xs, noisy, W1, b1, W2, b2, interpret=False):
    probs = pl.pallas_call(
        _router_body,
        out_shape=jax.ShapeDtypeStruct((S, E), jnp.float32),
        interpret=interpret,
    )(noisy)

    out = pl.pallas_call(
        _expert_body,
        grid=(E, NH),
        in_specs=[
            pl.BlockSpec((S, D), lambda e, j: (0, 0)),
            pl.BlockSpec((S, E), lambda e, j: (0, 0)),
            pl.BlockSpec((1, D, HC), lambda e, j: (e, 0, j)),
            pl.BlockSpec((1, 1, HC), lambda e, j: (e, 0, j)),
            pl.BlockSpec((1, HC, D), lambda e, j: (e, j, 0)),
            pl.BlockSpec((1, 1, D), lambda e, j: (e, 0, 0)),
        ],
        out_specs=pl.BlockSpec((S, D), lambda e, j: (0, 0)),
        out_shape=jax.ShapeDtypeStruct((S, D), jnp.float32),
        compiler_params=pltpu.CompilerParams(
            dimension_semantics=("arbitrary", "arbitrary"),
        ),
        interpret=interpret,
    )(xs, probs, W1, b1.reshape(E, 1, H), W2, b2.reshape(E, 1, D))
    return out


def kernel(x, Wg, bg, Wn, bn, W1, b1, W2, b2, noise):
    logits = x @ Wg + bg
    noise_logits = x @ Wn + bn
    noisy = logits + noise * jax.nn.softplus(noise_logits)
    out = _run(x[0], noisy[0], W1, b1, W2, b2)
    return out[None]


# speedup vs baseline: 2.8104x; 1.1503x over previous
import functools
import math

import jax
import jax.numpy as jnp
from jax import lax
from jax.experimental import pallas as pl
from jax.experimental.pallas import tpu as pltpu
from jax.experimental.pallas import tpu_sc as plsc

S, D, E, K, H = 2048, 1024, 8, 2, 4096
HC = 512
NH = H // HC
TM = 128
NPAIR = K * S
MAX_SLOTS = 5120
NC, NS = 2, 16
NW = NC * NS
CHUNK = 64
NCHUNK = NPAIR // (NW * CHUNK)

_HIGHEST = jax.lax.Precision.HIGHEST


def _gelu(v):
    return 0.5 * v * (1.0 + jax.lax.erf(v * (1.0 / math.sqrt(2.0))))


def _router_body(noisy_ref, p0_ref, p1_ref, pos0_ref, pos1_ref, nt_ref,
                 bs_ref):
    noisy = noisy_ref[...]
    lanes = jax.lax.broadcasted_iota(jnp.int32, (S, E), 1)

    v0 = jnp.max(noisy, axis=1, keepdims=True)
    e0 = jnp.min(jnp.where(noisy == v0, lanes, E), axis=1, keepdims=True)
    oh0 = lanes == e0
    masked = jnp.where(oh0, -jnp.inf, noisy)
    v1 = jnp.max(masked, axis=1, keepdims=True)
    e1 = jnp.min(jnp.where(masked == v1, lanes, E), axis=1, keepdims=True)
    oh1 = lanes == e1

    z = jnp.exp(v1 - v0)
    p0_ref[...] = 1.0 / (1.0 + z)
    p1_ref[...] = z / (1.0 + z)

    m = jnp.where(oh0 | oh1, 1.0, 0.0)
    r = jax.lax.broadcasted_iota(jnp.int32, (S, S), 0)
    c = jax.lax.broadcasted_iota(jnp.int32, (S, S), 1)
    tri = jnp.where(r > c, 1.0, 0.0)
    csum = jnp.dot(tri, m, preferred_element_type=jnp.float32,
                   precision=_HIGHEST)

    counts = jnp.sum(m, axis=0, keepdims=True)
    ntf = jnp.floor((counts + (TM - 1)) * (1.0 / TM))
    re = jax.lax.broadcasted_iota(jnp.int32, (E, E), 0)
    ce = jax.lax.broadcasted_iota(jnp.int32, (E, E), 1)
    trie = jnp.where(re < ce, 1.0, 0.0)
    bsf = jnp.dot(ntf, trie, preferred_element_type=jnp.float32,
                  precision=_HIGHEST) * float(TM)

    rank0 = jnp.sum(jnp.where(oh0, csum, 0.0), axis=1, keepdims=True)
    rank1 = jnp.sum(jnp.where(oh1, csum, 0.0), axis=1, keepdims=True)
    base0 = jnp.sum(jnp.where(oh0, bsf, 0.0), axis=1, keepdims=True)
    base1 = jnp.sum(jnp.where(oh1, bsf, 0.0), axis=1, keepdims=True)
    pos0_ref[...] = (base0 + rank0).astype(jnp.int32)
    pos1_ref[...] = (base1 + rank1).astype(jnp.int32)
    nt_ref[...] = ntf.astype(jnp.int32)
    bs_ref[...] = bsf.astype(jnp.int32)


def _router(noisy, interpret=False):
    return pl.pallas_call(
        _router_body,
        out_shape=(
            jax.ShapeDtypeStruct((S, 1), jnp.float32),
            jax.ShapeDtypeStruct((S, 1), jnp.float32),
            jax.ShapeDtypeStruct((S, 1), jnp.int32),
            jax.ShapeDtypeStruct((S, 1), jnp.int32),
            jax.ShapeDtypeStruct((1, E), jnp.int32),
            jax.ShapeDtypeStruct((1, E), jnp.int32),
        ),
        interpret=interpret,
    )(noisy)


def _sc_mesh():
    return plsc.VectorSubcoreMesh(core_axis_name="c", subcore_axis_name="s")


def _scatter_x(xs, pos01):
    @functools.partial(
        pl.kernel,
        out_type=jax.ShapeDtypeStruct((MAX_SLOTS, D), jnp.float32),
        mesh=_sc_mesh(),
        scratch_types=[
            pltpu.VMEM((CHUNK,), jnp.int32),
            pltpu.VMEM((CHUNK, D), jnp.float32),
            pltpu.SemaphoreType.DMA,
        ],
    )
    def k(x_hbm, pos_hbm, xg_hbm, idx_v, rows_v, sem):
        wid = lax.axis_index("s") * NC + lax.axis_index("c")
        for c in range(NCHUNK):
            base = wid * (NCHUNK * CHUNK) + c * CHUNK
            src = lax.rem(base, S)
            pltpu.sync_copy(pos_hbm.at[pl.ds(base, CHUNK)], idx_v)
            pltpu.sync_copy(x_hbm.at[pl.ds(src, CHUNK)], rows_v)
            pltpu.async_copy(rows_v, xg_hbm.at[idx_v], sem).wait()

    return k(xs, pos01)


def _gather_y(yg, pos01):
    @functools.partial(
        pl.kernel,
        out_type=jax.ShapeDtypeStruct((NPAIR, D), jnp.float32),
        mesh=_sc_mesh(),
        scratch_types=[
            pltpu.VMEM((CHUNK,), jnp.int32),
            pltpu.VMEM((CHUNK, D), jnp.float32),
            pltpu.SemaphoreType.DMA,
        ],
    )
    def k(yg_hbm, pos_hbm, out_hbm, idx_v, rows_v, sem):
        wid = lax.axis_index("s") * NC + lax.axis_index("c")
        for c in range(NCHUNK):
            base = wid * (NCHUNK * CHUNK) + c * CHUNK
            pltpu.sync_copy(pos_hbm.at[pl.ds(base, CHUNK)], idx_v)
            pltpu.async_copy(yg_hbm.at[idx_v], rows_v, sem).wait()
            pltpu.sync_copy(rows_v, out_hbm.at[pl.ds(base, CHUNK)])

    return k(yg, pos01)


def _expert_body(nt_ref, bs_ref, xg_ref, w1_ref, b1_ref, w2_ref, b2_ref,
                 yg_ref):
    e = pl.program_id(0)
    j = pl.program_id(1)
    w1 = w1_ref[0]
    w2 = w2_ref[0]
    b1 = b1_ref[0]
    b2 = b2_ref[0]
    base = bs_ref[e]

    def tile(i, _):
        sl = pl.ds(pl.multiple_of(base + i * TM, TM), TM)
        xt = xg_ref[sl, :]
        h = _gelu(jnp.dot(xt, w1, preferred_element_type=jnp.float32) + b1)
        contrib = jnp.dot(h, w2, preferred_element_type=jnp.float32)

        @pl.when(j == 0)
        def _():
            yg_ref[sl, :] = contrib + b2

        @pl.when(j > 0)
        def _():
            yg_ref[sl, :] = yg_ref[sl, :] + contrib

        return 0

    jax.lax.fori_loop(0, nt_ref[e], tile, 0)


def _experts(nt8, bs8, xg, W1, b1, W2, b2, interpret=False):
    return pl.pallas_call(
        _expert_body,
        grid_spec=pltpu.PrefetchScalarGridSpec(
            num_scalar_prefetch=2,
            grid=(E, NH),
            in_specs=[
                pl.BlockSpec((MAX_SLOTS, D), lambda e, j, nt, bs: (0, 0)),
                pl.BlockSpec((1, D, HC), lambda e, j, nt, bs: (e, 0, j)),
                pl.BlockSpec((1, 1, HC), lambda e, j, nt, bs: (e, 0, j)),
                pl.BlockSpec((1, HC, D), lambda e, j, nt, bs: (e, j, 0)),
                pl.BlockSpec((1, 1, D), lambda e, j, nt, bs: (e, 0, 0)),
            ],
            out_specs=pl.BlockSpec((MAX_SLOTS, D), lambda e, j, nt, bs: (0, 0)),
        ),
        out_shape=jax.ShapeDtypeStruct((MAX_SLOTS, D), jnp.float32),
        compiler_params=pltpu.CompilerParams(
            dimension_semantics=("arbitrary", "arbitrary"),
        ),
        interpret=interpret,
    )(nt8, bs8, xg, W1, b1.reshape(E, 1, H), W2, b2.reshape(E, 1, D))


def _combine_body(y01_ref, p0_ref, p1_ref, out_ref):
    out_ref[...] = (p0_ref[...] * y01_ref[0] + p1_ref[...] * y01_ref[1])


def _combine(y01, p0, p1, interpret=False):
    return pl.pallas_call(
        _combine_body,
        grid=(1,),
        in_specs=[
            pl.BlockSpec((2, S, D), lambda i: (0, 0, 0)),
            pl.BlockSpec((S, 1), lambda i: (0, 0)),
            pl.BlockSpec((S, 1), lambda i: (0, 0)),
        ],
        out_specs=pl.BlockSpec((S, D), lambda i: (0, 0)),
        out_shape=jax.ShapeDtypeStruct((S, D), jnp.float32),
        interpret=interpret,
    )(y01.reshape(2, S, D), p0, p1)


def _run(xs, noisy, W1, b1, W2, b2, interpret=False,
         scatter=_scatter_x, gather=_gather_y):
    p0, p1, pos0, pos1, nt8, bs8 = _router(noisy, interpret=interpret)
    pos01 = jnp.concatenate([pos0.reshape(-1), pos1.reshape(-1)])
    xg = scatter(xs, pos01)
    yg = _experts(nt8.reshape(-1), bs8.reshape(-1), xg, W1, b1, W2, b2,
                  interpret=interpret)
    y01 = gather(yg, pos01)
    return _combine(y01, p0, p1, interpret=interpret)


def kernel(x, Wg, bg, Wn, bn, W1, b1, W2, b2, noise):
    logits = x @ Wg + bg
    noise_logits = x @ Wn + bn
    noisy = logits + noise * jax.nn.softplus(noise_logits)
    out = _run(x[0], noisy[0], W1, b1, W2, b2)
    return out[None]


# speedup vs baseline: 2.9132x; 1.0366x over previous
import functools
import math

import jax
import jax.numpy as jnp
from jax import lax
from jax.experimental import pallas as pl
from jax.experimental.pallas import tpu as pltpu
from jax.experimental.pallas import tpu_sc as plsc

S, D, E, K, H = 2048, 1024, 8, 2, 4096
HC = 512
NH = H // HC
TM = 128
NPAIR = K * S
MAX_SLOTS = 5120
NC, NS = 2, 16
NW = NC * NS
CHUNK = 64
NCHUNK = NPAIR // (NW * CHUNK)

_HIGHEST = jax.lax.Precision.HIGHEST


def _gelu(v):
    return 0.5 * v * (1.0 + jax.lax.erf(v * (1.0 / math.sqrt(2.0))))


def _router_body(noisy_ref, p0_ref, p1_ref, pos0_ref, pos1_ref, nt_ref,
                 bs_ref):
    noisy = noisy_ref[...]
    lanes = jax.lax.broadcasted_iota(jnp.int32, (S, E), 1)

    v0 = jnp.max(noisy, axis=1, keepdims=True)
    e0 = jnp.min(jnp.where(noisy == v0, lanes, E), axis=1, keepdims=True)
    oh0 = lanes == e0
    masked = jnp.where(oh0, -jnp.inf, noisy)
    v1 = jnp.max(masked, axis=1, keepdims=True)
    e1 = jnp.min(jnp.where(masked == v1, lanes, E), axis=1, keepdims=True)
    oh1 = lanes == e1

    z = jnp.exp(v1 - v0)
    p0_ref[...] = 1.0 / (1.0 + z)
    p1_ref[...] = z / (1.0 + z)

    m = jnp.where(oh0 | oh1, 1.0, 0.0)
    r = jax.lax.broadcasted_iota(jnp.int32, (S, S), 0)
    c = jax.lax.broadcasted_iota(jnp.int32, (S, S), 1)
    tri = jnp.where(r > c, 1.0, 0.0)
    csum = jnp.dot(tri, m, preferred_element_type=jnp.float32)

    counts = jnp.sum(m, axis=0, keepdims=True)
    ntf = jnp.floor((counts + (TM - 1)) * (1.0 / TM))
    re = jax.lax.broadcasted_iota(jnp.int32, (E, E), 0)
    ce = jax.lax.broadcasted_iota(jnp.int32, (E, E), 1)
    trie = jnp.where(re < ce, 1.0, 0.0)
    bsf = jnp.dot(ntf, trie,
                  preferred_element_type=jnp.float32) * float(TM)

    rank0 = jnp.sum(jnp.where(oh0, csum, 0.0), axis=1, keepdims=True)
    rank1 = jnp.sum(jnp.where(oh1, csum, 0.0), axis=1, keepdims=True)
    base0 = jnp.sum(jnp.where(oh0, bsf, 0.0), axis=1, keepdims=True)
    base1 = jnp.sum(jnp.where(oh1, bsf, 0.0), axis=1, keepdims=True)
    pos0_ref[...] = (base0 + rank0).astype(jnp.int32)
    pos1_ref[...] = (base1 + rank1).astype(jnp.int32)
    nt_ref[...] = ntf.astype(jnp.int32)
    bs_ref[...] = bsf.astype(jnp.int32)


def _router(noisy, interpret=False):
    return pl.pallas_call(
        _router_body,
        out_shape=(
            jax.ShapeDtypeStruct((S, 1), jnp.float32),
            jax.ShapeDtypeStruct((S, 1), jnp.float32),
            jax.ShapeDtypeStruct((S, 1), jnp.int32),
            jax.ShapeDtypeStruct((S, 1), jnp.int32),
            jax.ShapeDtypeStruct((1, E), jnp.int32),
            jax.ShapeDtypeStruct((1, E), jnp.int32),
        ),
        interpret=interpret,
    )(noisy)


def _sc_mesh():
    return plsc.VectorSubcoreMesh(core_axis_name="c", subcore_axis_name="s")


def _scatter_x(xs, pos01):
    @functools.partial(
        pl.kernel,
        out_type=jax.ShapeDtypeStruct((MAX_SLOTS, D), jnp.float32),
        mesh=_sc_mesh(),
        scratch_types=[
            pltpu.VMEM((CHUNK,), jnp.int32),
            pltpu.VMEM((CHUNK, D), jnp.float32),
            pltpu.SemaphoreType.DMA,
        ],
    )
    def k(x_hbm, pos_hbm, xg_hbm, idx_v, rows_v, sem):
        wid = lax.axis_index("s") * NC + lax.axis_index("c")
        for c in range(NCHUNK):
            base = wid * (NCHUNK * CHUNK) + c * CHUNK
            src = lax.rem(base, S)
            pltpu.sync_copy(pos_hbm.at[pl.ds(base, CHUNK)], idx_v)
            pltpu.sync_copy(x_hbm.at[pl.ds(src, CHUNK)], rows_v)
            pltpu.async_copy(rows_v, xg_hbm.at[idx_v], sem).wait()

    return k(xs, pos01)


def _gather_y(yg, pos01):
    @functools.partial(
        pl.kernel,
        out_type=jax.ShapeDtypeStruct((NPAIR, D), jnp.float32),
        mesh=_sc_mesh(),
        scratch_types=[
            pltpu.VMEM((CHUNK,), jnp.int32),
            pltpu.VMEM((CHUNK, D), jnp.float32),
            pltpu.SemaphoreType.DMA,
        ],
    )
    def k(yg_hbm, pos_hbm, out_hbm, idx_v, rows_v, sem):
        wid = lax.axis_index("s") * NC + lax.axis_index("c")
        for c in range(NCHUNK):
            base = wid * (NCHUNK * CHUNK) + c * CHUNK
            pltpu.sync_copy(pos_hbm.at[pl.ds(base, CHUNK)], idx_v)
            pltpu.async_copy(yg_hbm.at[idx_v], rows_v, sem).wait()
            pltpu.sync_copy(rows_v, out_hbm.at[pl.ds(base, CHUNK)])

    return k(yg, pos01)


def _expert_body(nt_ref, bs_ref, xg_ref, w1_ref, b1_ref, w2_ref, b2_ref,
                 yg_ref):
    e = pl.program_id(0)
    j = pl.program_id(1)
    w1 = w1_ref[0]
    w2 = w2_ref[0]
    b1 = b1_ref[0]
    b2 = b2_ref[0]
    base = bs_ref[0, e]

    def tile(i, _):
        sl = pl.ds(pl.multiple_of(base + i * TM, TM), TM)
        xt = xg_ref[sl, :]
        h = _gelu(jnp.dot(xt, w1, preferred_element_type=jnp.float32) + b1)
        contrib = jnp.dot(h, w2, preferred_element_type=jnp.float32)

        @pl.when(j == 0)
        def _():
            yg_ref[sl, :] = contrib + b2

        @pl.when(j > 0)
        def _():
            yg_ref[sl, :] = yg_ref[sl, :] + contrib

        return 0

    jax.lax.fori_loop(0, nt_ref[0, e], tile, 0)


def _experts(nt8, bs8, xg, W1, b1, W2, b2, interpret=False):
    return pl.pallas_call(
        _expert_body,
        grid_spec=pltpu.PrefetchScalarGridSpec(
            num_scalar_prefetch=2,
            grid=(E, NH),
            in_specs=[
                pl.BlockSpec((MAX_SLOTS, D), lambda e, j, nt, bs: (0, 0)),
                pl.BlockSpec((1, D, HC), lambda e, j, nt, bs: (e, 0, j)),
                pl.BlockSpec((1, 1, HC), lambda e, j, nt, bs: (e, 0, j)),
                pl.BlockSpec((1, HC, D), lambda e, j, nt, bs: (e, j, 0)),
                pl.BlockSpec((1, 1, D), lambda e, j, nt, bs: (e, 0, 0)),
            ],
            out_specs=pl.BlockSpec((MAX_SLOTS, D), lambda e, j, nt, bs: (0, 0)),
        ),
        out_shape=jax.ShapeDtypeStruct((MAX_SLOTS, D), jnp.float32),
        compiler_params=pltpu.CompilerParams(
            dimension_semantics=("arbitrary", "arbitrary"),
        ),
        interpret=interpret,
    )(nt8, bs8, xg, W1, b1.reshape(E, 1, H), W2, b2.reshape(E, 1, D))


def _combine_body(y01_ref, p0_ref, p1_ref, out_ref):
    out_ref[...] = (p0_ref[...] * y01_ref[0] + p1_ref[...] * y01_ref[1])


def _combine(y01, p0, p1, interpret=False):
    return pl.pallas_call(
        _combine_body,
        grid=(1,),
        in_specs=[
            pl.BlockSpec((2, S, D), lambda i: (0, 0, 0)),
            pl.BlockSpec((S, 1), lambda i: (0, 0)),
            pl.BlockSpec((S, 1), lambda i: (0, 0)),
        ],
        out_specs=pl.BlockSpec((S, D), lambda i: (0, 0)),
        out_shape=jax.ShapeDtypeStruct((S, D), jnp.float32),
        interpret=interpret,
    )(y01.reshape(2, S, D), p0, p1)


def _run(xs, noisy, W1, b1, W2, b2, interpret=False,
         scatter=_scatter_x, gather=_gather_y):
    p0, p1, pos0, pos1, nt8, bs8 = _router(noisy, interpret=interpret)
    pos01 = jnp.concatenate([pos0.reshape(-1), pos1.reshape(-1)])
    xg = scatter(xs, pos01)
    yg = _experts(nt8, bs8, xg, W1, b1, W2, b2, interpret=interpret)
    y01 = gather(yg, pos01)
    return _combine(y01, p0, p1, interpret=interpret)


def kernel(x, Wg, bg, Wn, bn, W1, b1, W2, b2, noise):
    logits = x @ Wg + bg
    noise_logits = x @ Wn + bn
    noisy = logits + noise * jax.nn.softplus(noise_logits)
    out = _run(x[0], noisy[0], W1, b1, W2, b2)
    return out[None]


# speedup vs baseline: 3.1689x; 1.0878x over previous
import functools
import math

import jax
import jax.numpy as jnp
from jax import lax
from jax.experimental import pallas as pl
from jax.experimental.pallas import tpu as pltpu
from jax.experimental.pallas import tpu_sc as plsc

S, D, E, K, H = 2048, 1024, 8, 2, 4096
HC = 512
NH = H // HC
TM = 128
NPAIR = K * S
MAX_SLOTS = 5120
MAX_TILES = MAX_SLOTS // TM
NC, NS = 2, 16
NW = NC * NS
CHUNK = 64
NCHUNK = NPAIR // (NW * CHUNK)

_HIGHEST = jax.lax.Precision.HIGHEST


def _gelu(v):
    return 0.5 * v * (1.0 + jax.lax.erf(v * (1.0 / math.sqrt(2.0))))


def _router_body(noisy_ref, p0_ref, p1_ref, pos0_ref, pos1_ref, texp_ref):
    noisy = noisy_ref[...]
    lanes = jax.lax.broadcasted_iota(jnp.int32, (S, E), 1)

    v0 = jnp.max(noisy, axis=1, keepdims=True)
    e0 = jnp.min(jnp.where(noisy == v0, lanes, E), axis=1, keepdims=True)
    oh0 = lanes == e0
    masked = jnp.where(oh0, -jnp.inf, noisy)
    v1 = jnp.max(masked, axis=1, keepdims=True)
    e1 = jnp.min(jnp.where(masked == v1, lanes, E), axis=1, keepdims=True)
    oh1 = lanes == e1

    z = jnp.exp(v1 - v0)
    p0_ref[...] = 1.0 / (1.0 + z)
    p1_ref[...] = z / (1.0 + z)

    m = jnp.where(oh0 | oh1, 1.0, 0.0)
    r = jax.lax.broadcasted_iota(jnp.int32, (S, S), 0)
    c = jax.lax.broadcasted_iota(jnp.int32, (S, S), 1)
    tri = jnp.where(r > c, 1.0, 0.0)
    csum = jnp.dot(tri, m, preferred_element_type=jnp.float32)

    counts = jnp.sum(m, axis=0, keepdims=True)
    ntf = jnp.floor((counts + (TM - 1)) * (1.0 / TM))
    re = jax.lax.broadcasted_iota(jnp.int32, (E, E), 0)
    ce = jax.lax.broadcasted_iota(jnp.int32, (E, E), 1)
    trie = jnp.where(re < ce, 1.0, 0.0)
    btile = jnp.dot(ntf, trie,
                    preferred_element_type=jnp.float32)
    bsf = btile * float(TM)

    rank0 = jnp.sum(jnp.where(oh0, csum, 0.0), axis=1, keepdims=True)
    rank1 = jnp.sum(jnp.where(oh1, csum, 0.0), axis=1, keepdims=True)
    base0 = jnp.sum(jnp.where(oh0, bsf, 0.0), axis=1, keepdims=True)
    base1 = jnp.sum(jnp.where(oh1, bsf, 0.0), axis=1, keepdims=True)
    pos0_ref[...] = (base0 + rank0).astype(jnp.int32)
    pos1_ref[...] = (base1 + rank1).astype(jnp.int32)

    bt_col = jnp.reshape(btile, (E, 1))
    ti = jax.lax.broadcasted_iota(jnp.int32, (E, MAX_TILES), 1)
    texp = jnp.sum(jnp.where(bt_col <= ti.astype(jnp.float32), 1.0, 0.0),
                   axis=0, keepdims=True) - 1.0
    texp_ref[...] = texp.astype(jnp.int32)


def _router(noisy, interpret=False):
    return pl.pallas_call(
        _router_body,
        out_shape=(
            jax.ShapeDtypeStruct((S, 1), jnp.float32),
            jax.ShapeDtypeStruct((S, 1), jnp.float32),
            jax.ShapeDtypeStruct((S, 1), jnp.int32),
            jax.ShapeDtypeStruct((S, 1), jnp.int32),
            jax.ShapeDtypeStruct((1, MAX_TILES), jnp.int32),
        ),
        interpret=interpret,
    )(noisy)


def _sc_mesh():
    return plsc.VectorSubcoreMesh(core_axis_name="c", subcore_axis_name="s")


def _scatter_x(xs, pos01):
    @functools.partial(
        pl.kernel,
        out_type=jax.ShapeDtypeStruct((MAX_SLOTS, D), jnp.float32),
        mesh=_sc_mesh(),
        scratch_types=[
            pltpu.VMEM((CHUNK,), jnp.int32),
            pltpu.VMEM((CHUNK, D), jnp.float32),
            pltpu.SemaphoreType.DMA,
        ],
    )
    def k(x_hbm, pos_hbm, xg_hbm, idx_v, rows_v, sem):
        wid = lax.axis_index("s") * NC + lax.axis_index("c")
        for c in range(NCHUNK):
            base = wid * (NCHUNK * CHUNK) + c * CHUNK
            src = lax.rem(base, S)
            pltpu.sync_copy(pos_hbm.at[pl.ds(base, CHUNK)], idx_v)
            pltpu.sync_copy(x_hbm.at[pl.ds(src, CHUNK)], rows_v)
            pltpu.async_copy(rows_v, xg_hbm.at[idx_v], sem).wait()

    return k(xs, pos01)


def _gather_y(yg, pos01):
    @functools.partial(
        pl.kernel,
        out_type=jax.ShapeDtypeStruct((NPAIR, D), jnp.float32),
        mesh=_sc_mesh(),
        scratch_types=[
            pltpu.VMEM((CHUNK,), jnp.int32),
            pltpu.VMEM((CHUNK, D), jnp.float32),
            pltpu.SemaphoreType.DMA,
        ],
    )
    def k(yg_hbm, pos_hbm, out_hbm, idx_v, rows_v, sem):
        wid = lax.axis_index("s") * NC + lax.axis_index("c")
        for c in range(NCHUNK):
            base = wid * (NCHUNK * CHUNK) + c * CHUNK
            pltpu.sync_copy(pos_hbm.at[pl.ds(base, CHUNK)], idx_v)
            pltpu.async_copy(yg_hbm.at[idx_v], rows_v, sem).wait()
            pltpu.sync_copy(rows_v, out_hbm.at[pl.ds(base, CHUNK)])

    return k(yg, pos01)


def _fc1_body(texp_ref, xg_ref, w1_ref, b1_ref, hg_ref):
    h = jnp.dot(xg_ref[...], w1_ref[0], preferred_element_type=jnp.float32)
    hg_ref[...] = _gelu(h + b1_ref[0])


def _fc1(texp, xg, W1, b1, interpret=False):
    return pl.pallas_call(
        _fc1_body,
        grid_spec=pltpu.PrefetchScalarGridSpec(
            num_scalar_prefetch=1,
            grid=(MAX_TILES,),
            in_specs=[
                pl.BlockSpec((TM, D), lambda i, texp: (i, 0)),
                pl.BlockSpec((1, D, H), lambda i, texp: (texp[0, i], 0, 0)),
                pl.BlockSpec((1, 1, H), lambda i, texp: (texp[0, i], 0, 0)),
            ],
            out_specs=pl.BlockSpec((TM, H), lambda i, texp: (i, 0)),
        ),
        out_shape=jax.ShapeDtypeStruct((MAX_SLOTS, H), jnp.float32),
        compiler_params=pltpu.CompilerParams(
            dimension_semantics=("arbitrary",),
        ),
        interpret=interpret,
    )(texp, xg, W1, b1.reshape(E, 1, H))


def _fc2_body(texp_ref, hg_ref, w2_ref, b2_ref, yg_ref):
    y = jnp.dot(hg_ref[...], w2_ref[0], preferred_element_type=jnp.float32)
    yg_ref[...] = y + b2_ref[0]


def _fc2(texp, hg, W2, b2, interpret=False):
    return pl.pallas_call(
        _fc2_body,
        grid_spec=pltpu.PrefetchScalarGridSpec(
            num_scalar_prefetch=1,
            grid=(MAX_TILES,),
            in_specs=[
                pl.BlockSpec((TM, H), lambda i, texp: (i, 0)),
                pl.BlockSpec((1, H, D), lambda i, texp: (texp[0, i], 0, 0)),
                pl.BlockSpec((1, 1, D), lambda i, texp: (texp[0, i], 0, 0)),
            ],
            out_specs=pl.BlockSpec((TM, D), lambda i, texp: (i, 0)),
        ),
        out_shape=jax.ShapeDtypeStruct((MAX_SLOTS, D), jnp.float32),
        compiler_params=pltpu.CompilerParams(
            dimension_semantics=("arbitrary",),
        ),
        interpret=interpret,
    )(texp, hg, W2, b2.reshape(E, 1, D))


def _combine_body(y01_ref, p0_ref, p1_ref, out_ref):
    out_ref[...] = (p0_ref[...] * y01_ref[0] + p1_ref[...] * y01_ref[1])


def _combine(y01, p0, p1, interpret=False):
    return pl.pallas_call(
        _combine_body,
        grid=(1,),
        in_specs=[
            pl.BlockSpec((2, S, D), lambda i: (0, 0, 0)),
            pl.BlockSpec((S, 1), lambda i: (0, 0)),
            pl.BlockSpec((S, 1), lambda i: (0, 0)),
        ],
        out_specs=pl.BlockSpec((S, D), lambda i: (0, 0)),
        out_shape=jax.ShapeDtypeStruct((S, D), jnp.float32),
        interpret=interpret,
    )(y01.reshape(2, S, D), p0, p1)


def _run(xs, noisy, W1, b1, W2, b2, interpret=False,
         scatter=_scatter_x, gather=_gather_y):
    p0, p1, pos0, pos1, texp = _router(noisy, interpret=interpret)
    pos01 = jnp.concatenate([pos0.reshape(-1), pos1.reshape(-1)])
    xg = scatter(xs, pos01)
    hg = _fc1(texp, xg, W1, b1, interpret=interpret)
    yg = _fc2(texp, hg, W2, b2, interpret=interpret)
    y01 = gather(yg, pos01)
    return _combine(y01, p0, p1, interpret=interpret)


def kernel(x, Wg, bg, Wn, bn, W1, b1, W2, b2, noise):
    logits = x @ Wg + bg
    noise_logits = x @ Wn + bn
    noisy = logits + noise * jax.nn.softplus(noise_logits)
    out = _run(x[0], noisy[0], W1, b1, W2, b2)
    return out[None]


# speedup vs baseline: 3.2649x; 1.0303x over previous
import functools
import math

import jax
import jax.numpy as jnp
from jax import lax
from jax.experimental import pallas as pl
from jax.experimental.pallas import tpu as pltpu
from jax.experimental.pallas import tpu_sc as plsc

S, D, E, K, H = 2048, 1024, 8, 2, 4096
HC = 512
NH = H // HC
TM = 128
NPAIR = K * S
MAX_SLOTS = 5120
MAX_TILES = MAX_SLOTS // TM
D2 = D // 2
NC, NS = 2, 16
NW = NC * NS
CHUNK = 64
NCHUNK = NPAIR // (NW * CHUNK)

_HIGHEST = jax.lax.Precision.HIGHEST


def _gelu(v):
    return 0.5 * v * (1.0 + jax.lax.erf(v * (1.0 / math.sqrt(2.0))))


def _router_body(noisy_ref, p0_ref, p1_ref, pos0_ref, pos1_ref, texp_ref):
    noisy = noisy_ref[...]
    lanes = jax.lax.broadcasted_iota(jnp.int32, (S, E), 1)

    v0 = jnp.max(noisy, axis=1, keepdims=True)
    e0 = jnp.min(jnp.where(noisy == v0, lanes, E), axis=1, keepdims=True)
    oh0 = lanes == e0
    masked = jnp.where(oh0, -jnp.inf, noisy)
    v1 = jnp.max(masked, axis=1, keepdims=True)
    e1 = jnp.min(jnp.where(masked == v1, lanes, E), axis=1, keepdims=True)
    oh1 = lanes == e1

    z = jnp.exp(v1 - v0)
    p0_ref[...] = 1.0 / (1.0 + z)
    p1_ref[...] = z / (1.0 + z)

    m = jnp.where(oh0 | oh1, 1.0, 0.0)
    r = jax.lax.broadcasted_iota(jnp.int32, (S, S), 0)
    c = jax.lax.broadcasted_iota(jnp.int32, (S, S), 1)
    tri = jnp.where(r > c, 1.0, 0.0)
    csum = jnp.dot(tri, m, preferred_element_type=jnp.float32)

    counts = jnp.sum(m, axis=0, keepdims=True)
    ntf = jnp.floor((counts + (TM - 1)) * (1.0 / TM))
    re = jax.lax.broadcasted_iota(jnp.int32, (E, E), 0)
    ce = jax.lax.broadcasted_iota(jnp.int32, (E, E), 1)
    trie = jnp.where(re < ce, 1.0, 0.0)
    btile = jnp.dot(ntf, trie,
                    preferred_element_type=jnp.float32)
    bsf = btile * float(TM)

    rank0 = jnp.sum(jnp.where(oh0, csum, 0.0), axis=1, keepdims=True)
    rank1 = jnp.sum(jnp.where(oh1, csum, 0.0), axis=1, keepdims=True)
    base0 = jnp.sum(jnp.where(oh0, bsf, 0.0), axis=1, keepdims=True)
    base1 = jnp.sum(jnp.where(oh1, bsf, 0.0), axis=1, keepdims=True)
    pos0_ref[...] = (base0 + rank0).astype(jnp.int32)
    pos1_ref[...] = (base1 + rank1).astype(jnp.int32)

    bt_col = jnp.reshape(btile, (E, 1))
    ti = jax.lax.broadcasted_iota(jnp.int32, (E, MAX_TILES), 1)
    texp = jnp.sum(jnp.where(bt_col <= ti.astype(jnp.float32), 1.0, 0.0),
                   axis=0, keepdims=True) - 1.0
    texp_ref[...] = texp.astype(jnp.int32)


def _router(noisy, interpret=False):
    return pl.pallas_call(
        _router_body,
        out_shape=(
            jax.ShapeDtypeStruct((S, 1), jnp.float32),
            jax.ShapeDtypeStruct((S, 1), jnp.float32),
            jax.ShapeDtypeStruct((S, 1), jnp.int32),
            jax.ShapeDtypeStruct((S, 1), jnp.int32),
            jax.ShapeDtypeStruct((1, MAX_TILES), jnp.int32),
        ),
        interpret=interpret,
    )(noisy)


def _sc_mesh():
    return plsc.VectorSubcoreMesh(core_axis_name="c", subcore_axis_name="s")


def _scatter_x(xs, pos01):
    @functools.partial(
        pl.kernel,
        out_type=jax.ShapeDtypeStruct((MAX_SLOTS, D), jnp.float32),
        mesh=_sc_mesh(),
        scratch_types=[
            pltpu.VMEM((CHUNK,), jnp.int32),
            pltpu.VMEM((CHUNK, D), jnp.float32),
            pltpu.SemaphoreType.DMA,
        ],
    )
    def k(x_hbm, pos_hbm, xg_hbm, idx_v, rows_v, sem):
        wid = lax.axis_index("s") * NC + lax.axis_index("c")
        for c in range(NCHUNK):
            base = wid * (NCHUNK * CHUNK) + c * CHUNK
            src = lax.rem(base, S)
            pltpu.sync_copy(pos_hbm.at[pl.ds(base, CHUNK)], idx_v)
            pltpu.sync_copy(x_hbm.at[pl.ds(src, CHUNK)], rows_v)
            pltpu.async_copy(rows_v, xg_hbm.at[idx_v], sem).wait()

    return k(xs, pos01)


def _gather_y(yg, pos01):
    @functools.partial(
        pl.kernel,
        out_type=jax.ShapeDtypeStruct((NPAIR, D), jnp.float32),
        mesh=_sc_mesh(),
        scratch_types=[
            pltpu.VMEM((CHUNK,), jnp.int32),
            pltpu.VMEM((CHUNK, D), jnp.float32),
            pltpu.SemaphoreType.DMA,
        ],
    )
    def k(yg_hbm, pos_hbm, out_hbm, idx_v, rows_v, sem):
        wid = lax.axis_index("s") * NC + lax.axis_index("c")
        for c in range(NCHUNK):
            base = wid * (NCHUNK * CHUNK) + c * CHUNK
            pltpu.sync_copy(pos_hbm.at[pl.ds(base, CHUNK)], idx_v)
            pltpu.async_copy(yg_hbm.at[idx_v], rows_v, sem).wait()
            pltpu.sync_copy(rows_v, out_hbm.at[pl.ds(base, CHUNK)])

    return k(yg, pos01)


def _fc1_body(texp_ref, xg_ref, w1_ref, b1_ref, hg_ref):
    h = jnp.dot(xg_ref[...], w1_ref[0], preferred_element_type=jnp.float32)
    hg_ref[...] = _gelu(h + b1_ref[0]).astype(jnp.bfloat16)


def _fc1(texp, xg, W1, b1, interpret=False):
    return pl.pallas_call(
        _fc1_body,
        grid_spec=pltpu.PrefetchScalarGridSpec(
            num_scalar_prefetch=1,
            grid=(MAX_TILES,),
            in_specs=[
                pl.BlockSpec((TM, D), lambda i, texp: (i, 0)),
                pl.BlockSpec((1, D, H), lambda i, texp: (texp[0, i], 0, 0)),
                pl.BlockSpec((1, 1, H), lambda i, texp: (texp[0, i], 0, 0)),
            ],
            out_specs=pl.BlockSpec((TM, H), lambda i, texp: (i, 0)),
        ),
        out_shape=jax.ShapeDtypeStruct((MAX_SLOTS, H), jnp.bfloat16),
        compiler_params=pltpu.CompilerParams(
            dimension_semantics=("arbitrary",),
        ),
        interpret=interpret,
    )(texp, xg, W1, b1.reshape(E, 1, H))


def _fc2_body(texp_ref, hg_ref, w2_ref, b2_ref, yg_ref):
    y = jnp.dot(hg_ref[...].astype(jnp.float32), w2_ref[0],
                preferred_element_type=jnp.float32)
    yg_ref[...] = y + b2_ref[0]


def _fc2(texp, hg, W2, b2, interpret=False):
    return pl.pallas_call(
        _fc2_body,
        grid_spec=pltpu.PrefetchScalarGridSpec(
            num_scalar_prefetch=1,
            grid=(MAX_TILES,),
            in_specs=[
                pl.BlockSpec((TM, H), lambda i, texp: (i, 0)),
                pl.BlockSpec((1, H, D), lambda i, texp: (texp[0, i], 0, 0)),
                pl.BlockSpec((1, 1, D), lambda i, texp: (texp[0, i], 0, 0)),
            ],
            out_specs=pl.BlockSpec((TM, D), lambda i, texp: (i, 0)),
        ),
        out_shape=jax.ShapeDtypeStruct((MAX_SLOTS, D), jnp.float32),
        compiler_params=pltpu.CompilerParams(
            dimension_semantics=("arbitrary",),
        ),
        interpret=interpret,
    )(texp, hg, W2, b2.reshape(E, 1, D))


def _combine_body(y01_ref, p0_ref, p1_ref, out_ref):
    out_ref[...] = p0_ref[...] * y01_ref[0] + p1_ref[...] * y01_ref[1]


def _combine(y01, p0, p1, interpret=False):
    return pl.pallas_call(
        _combine_body,
        grid=(1,),
        in_specs=[
            pl.BlockSpec((2, S, D), lambda i: (0, 0, 0)),
            pl.BlockSpec((S, 1), lambda i: (0, 0)),
            pl.BlockSpec((S, 1), lambda i: (0, 0)),
        ],
        out_specs=pl.BlockSpec((S, D), lambda i: (0, 0)),
        out_shape=jax.ShapeDtypeStruct((S, D), jnp.float32),
        interpret=interpret,
    )(y01.reshape(2, S, D), p0, p1)


def _run(xs, noisy, W1, b1, W2, b2, interpret=False,
         scatter=_scatter_x, gather=_gather_y):
    p0, p1, pos0, pos1, texp = _router(noisy, interpret=interpret)
    pos01 = jnp.concatenate([pos0.reshape(-1), pos1.reshape(-1)])
    xg = scatter(xs, pos01)
    hg = _fc1(texp, xg, W1, b1, interpret=interpret)
    yg = _fc2(texp, hg, W2, b2, interpret=interpret)
    y01 = gather(yg, pos01)
    return _combine(y01, p0, p1, interpret=interpret)


def kernel(x, Wg, bg, Wn, bn, W1, b1, W2, b2, noise):
    logits = x @ Wg + bg
    noise_logits = x @ Wn + bn
    noisy = logits + noise * jax.nn.softplus(noise_logits)
    out = _run(x[0], noisy[0], W1, b1, W2, b2)
    return out[None]


# speedup vs baseline: 3.3958x; 1.0401x over previous
import functools
import math

import jax
import jax.numpy as jnp
from jax import lax
from jax.experimental import pallas as pl
from jax.experimental.pallas import tpu as pltpu
from jax.experimental.pallas import tpu_sc as plsc

S, D, E, K, H = 2048, 1024, 8, 2, 4096
HC = 512
NH = H // HC
TM = 128
NPAIR = K * S
MAX_SLOTS = 5120
MAX_TILES = MAX_SLOTS // TM
D2 = D // 2
NC, NS = 2, 16
NW = NC * NS
CHUNK = 64
NCHUNK = NPAIR // (NW * CHUNK)

_HIGHEST = jax.lax.Precision.HIGHEST


def _gelu(v):
    return 0.5 * v * (1.0 + jax.lax.erf(v * (1.0 / math.sqrt(2.0))))


NRUN = 16


def _router_body(noisy_ref, p0_ref, p1_ref, pos0_ref, pos1_ref, texp_ref,
                 runid_ref, rexp_ref):
    noisy = noisy_ref[...]
    lanes = jax.lax.broadcasted_iota(jnp.int32, (S, E), 1)

    v0 = jnp.max(noisy, axis=1, keepdims=True)
    e0 = jnp.min(jnp.where(noisy == v0, lanes, E), axis=1, keepdims=True)
    oh0 = lanes == e0
    masked = jnp.where(oh0, -jnp.inf, noisy)
    v1 = jnp.max(masked, axis=1, keepdims=True)
    e1 = jnp.min(jnp.where(masked == v1, lanes, E), axis=1, keepdims=True)
    oh1 = lanes == e1

    z = jnp.exp(v1 - v0)
    p0_ref[...] = 1.0 / (1.0 + z)
    p1_ref[...] = z / (1.0 + z)

    m = jnp.where(oh0 | oh1, 1.0, 0.0)
    r = jax.lax.broadcasted_iota(jnp.int32, (S, S), 0)
    c = jax.lax.broadcasted_iota(jnp.int32, (S, S), 1)
    tri = jnp.where(r > c, 1.0, 0.0)
    csum = jnp.dot(tri, m, preferred_element_type=jnp.float32)

    counts = jnp.sum(m, axis=0, keepdims=True)
    ntf = jnp.floor((counts + (TM - 1)) * (1.0 / TM))
    re = jax.lax.broadcasted_iota(jnp.int32, (E, E), 0)
    ce = jax.lax.broadcasted_iota(jnp.int32, (E, E), 1)
    trie = jnp.where(re < ce, 1.0, 0.0)
    btile = jnp.dot(ntf, trie,
                    preferred_element_type=jnp.float32)
    bsf = btile * float(TM)

    rank0 = jnp.sum(jnp.where(oh0, csum, 0.0), axis=1, keepdims=True)
    rank1 = jnp.sum(jnp.where(oh1, csum, 0.0), axis=1, keepdims=True)
    base0 = jnp.sum(jnp.where(oh0, bsf, 0.0), axis=1, keepdims=True)
    base1 = jnp.sum(jnp.where(oh1, bsf, 0.0), axis=1, keepdims=True)
    pos0_ref[...] = (base0 + rank0).astype(jnp.int32)
    pos1_ref[...] = (base1 + rank1).astype(jnp.int32)

    bt_col = jnp.reshape(btile, (E, 1))
    ti = jax.lax.broadcasted_iota(jnp.int32, (E, MAX_TILES), 1)
    texp = jnp.sum(jnp.where(bt_col <= ti.astype(jnp.float32), 1.0, 0.0),
                   axis=0, keepdims=True) - 1.0
    texp_ref[...] = texp.astype(jnp.int32)

    present_col = jnp.reshape(jnp.where(counts > 0.0, 1.0, 0.0), (E, 1))
    runidx_row = jnp.dot(jnp.reshape(present_col, (1, E)), trie,
                         preferred_element_type=jnp.float32)
    runidx_col = jnp.reshape(runidx_row, (E, 1))
    eid = jax.lax.broadcasted_iota(jnp.int32, (E, MAX_TILES), 0)
    sel = eid.astype(jnp.float32) == texp
    runid = jnp.sum(jnp.where(sel, runidx_col, 0.0), axis=0, keepdims=True)
    runid_ref[...] = runid.astype(jnp.int32)
    rr = jax.lax.broadcasted_iota(jnp.int32, (E, NRUN), 1).astype(jnp.float32)
    selr = jnp.logical_and(runidx_col == rr, present_col > 0.0)
    eidr = jax.lax.broadcasted_iota(jnp.int32, (E, NRUN), 0).astype(jnp.float32)
    rexp = jnp.sum(jnp.where(selr, eidr, 0.0), axis=0, keepdims=True)
    rexp_ref[...] = rexp.astype(jnp.int32)


def _router(noisy, interpret=False):
    return pl.pallas_call(
        _router_body,
        out_shape=(
            jax.ShapeDtypeStruct((S, 1), jnp.float32),
            jax.ShapeDtypeStruct((S, 1), jnp.float32),
            jax.ShapeDtypeStruct((S, 1), jnp.int32),
            jax.ShapeDtypeStruct((S, 1), jnp.int32),
            jax.ShapeDtypeStruct((1, MAX_TILES), jnp.int32),
            jax.ShapeDtypeStruct((1, MAX_TILES), jnp.int32),
            jax.ShapeDtypeStruct((1, NRUN), jnp.int32),
        ),
        interpret=interpret,
    )(noisy)


def _sc_mesh():
    return plsc.VectorSubcoreMesh(core_axis_name="c", subcore_axis_name="s")


def _scatter_x(xs, pos01):
    @functools.partial(
        pl.kernel,
        out_type=jax.ShapeDtypeStruct((MAX_SLOTS, D), jnp.float32),
        mesh=_sc_mesh(),
        scratch_types=[
            pltpu.VMEM((CHUNK,), jnp.int32),
            pltpu.VMEM((CHUNK, D), jnp.float32),
            pltpu.SemaphoreType.DMA,
        ],
    )
    def k(x_hbm, pos_hbm, xg_hbm, idx_v, rows_v, sem):
        wid = lax.axis_index("s") * NC + lax.axis_index("c")
        for c in range(NCHUNK):
            base = wid * (NCHUNK * CHUNK) + c * CHUNK
            src = lax.rem(base, S)
            pltpu.sync_copy(pos_hbm.at[pl.ds(base, CHUNK)], idx_v)
            pltpu.sync_copy(x_hbm.at[pl.ds(src, CHUNK)], rows_v)
            pltpu.async_copy(rows_v, xg_hbm.at[idx_v], sem).wait()

    return k(xs, pos01)


def _gather_y(yg, pos01):
    @functools.partial(
        pl.kernel,
        out_type=jax.ShapeDtypeStruct((NPAIR, D), jnp.float32),
        mesh=_sc_mesh(),
        scratch_types=[
            pltpu.VMEM((CHUNK,), jnp.int32),
            pltpu.VMEM((CHUNK, D), jnp.float32),
            pltpu.SemaphoreType.DMA,
        ],
    )
    def k(yg_hbm, pos_hbm, out_hbm, idx_v, rows_v, sem):
        wid = lax.axis_index("s") * NC + lax.axis_index("c")
        for c in range(NCHUNK):
            base = wid * (NCHUNK * CHUNK) + c * CHUNK
            pltpu.sync_copy(pos_hbm.at[pl.ds(base, CHUNK)], idx_v)
            pltpu.async_copy(yg_hbm.at[idx_v], rows_v, sem).wait()
            pltpu.sync_copy(rows_v, out_hbm.at[pl.ds(base, CHUNK)])

    return k(yg, pos01)


def _weight_prefetch(i, runid_ref, rexp_ref, w_hbm, wbuf, sems):
    rid = runid_ref[0, i]
    prev = runid_ref[0, jnp.maximum(i - 1, 0)]
    is_first = jnp.logical_or(i == 0, rid != prev)
    slot = jax.lax.rem(rid, 2)

    @pl.when(i == 0)
    def _():
        pltpu.make_async_copy(w_hbm.at[rexp_ref[0, 0]], wbuf.at[0],
                              sems.at[0]).start()
        pltpu.make_async_copy(w_hbm.at[rexp_ref[0, 1]], wbuf.at[1],
                              sems.at[1]).start()

    @pl.when(jnp.logical_and(is_first, rid >= 1))
    def _():
        pltpu.make_async_copy(w_hbm.at[rexp_ref[0, rid + 1]],
                              wbuf.at[1 - slot], sems.at[1 - slot]).start()

    @pl.when(is_first)
    def _():
        pltpu.make_async_copy(w_hbm.at[rexp_ref[0, rid]], wbuf.at[slot],
                              sems.at[slot]).wait()

    @pl.when(i == MAX_TILES - 1)
    def _():
        pltpu.make_async_copy(w_hbm.at[rexp_ref[0, rid + 1]],
                              wbuf.at[1 - slot], sems.at[1 - slot]).wait()

    return slot


def _fc1_body(texp_ref, runid_ref, rexp_ref, xg_ref, b1_ref, w1_hbm, hg_ref,
              wbuf, sems):
    i = pl.program_id(0)
    slot = _weight_prefetch(i, runid_ref, rexp_ref, w1_hbm, wbuf, sems)
    h = jnp.dot(xg_ref[...], wbuf[slot], preferred_element_type=jnp.float32)
    hg_ref[...] = _gelu(h + b1_ref[0]).astype(jnp.bfloat16)


def _fc1(texp, runid, rexp, xg, W1, b1, interpret=False):
    return pl.pallas_call(
        _fc1_body,
        grid_spec=pltpu.PrefetchScalarGridSpec(
            num_scalar_prefetch=3,
            grid=(MAX_TILES,),
            in_specs=[
                pl.BlockSpec((TM, D), lambda i, texp, runid, rexp: (i, 0)),
                pl.BlockSpec((1, 1, H),
                             lambda i, texp, runid, rexp: (texp[0, i], 0, 0)),
                pl.BlockSpec(memory_space=pl.ANY),
            ],
            out_specs=pl.BlockSpec((TM, H), lambda i, texp, runid, rexp: (i, 0)),
            scratch_shapes=[
                pltpu.VMEM((2, D, H), jnp.float32),
                pltpu.SemaphoreType.DMA((2,)),
            ],
        ),
        out_shape=jax.ShapeDtypeStruct((MAX_SLOTS, H), jnp.bfloat16),
        compiler_params=pltpu.CompilerParams(
            dimension_semantics=("arbitrary",),
        ),
        interpret=interpret,
    )(texp, runid, rexp, xg, b1.reshape(E, 1, H), W1)


def _fc2_body(texp_ref, runid_ref, rexp_ref, hg_ref, b2_ref, w2_hbm, yg_ref,
              wbuf, sems):
    i = pl.program_id(0)
    slot = _weight_prefetch(i, runid_ref, rexp_ref, w2_hbm, wbuf, sems)
    y = jnp.dot(hg_ref[...].astype(jnp.float32), wbuf[slot],
                preferred_element_type=jnp.float32)
    yg_ref[...] = y + b2_ref[0]


def _fc2(texp, runid, rexp, hg, W2, b2, interpret=False):
    return pl.pallas_call(
        _fc2_body,
        grid_spec=pltpu.PrefetchScalarGridSpec(
            num_scalar_prefetch=3,
            grid=(MAX_TILES,),
            in_specs=[
                pl.BlockSpec((TM, H), lambda i, texp, runid, rexp: (i, 0)),
                pl.BlockSpec((1, 1, D),
                             lambda i, texp, runid, rexp: (texp[0, i], 0, 0)),
                pl.BlockSpec(memory_space=pl.ANY),
            ],
            out_specs=pl.BlockSpec((TM, D), lambda i, texp, runid, rexp: (i, 0)),
            scratch_shapes=[
                pltpu.VMEM((2, H, D), jnp.float32),
                pltpu.SemaphoreType.DMA((2,)),
            ],
        ),
        out_shape=jax.ShapeDtypeStruct((MAX_SLOTS, D), jnp.float32),
        compiler_params=pltpu.CompilerParams(
            dimension_semantics=("arbitrary",),
        ),
        interpret=interpret,
    )(texp, runid, rexp, hg, b2.reshape(E, 1, D), W2)


def _combine_body(y01_ref, p0_ref, p1_ref, out_ref):
    out_ref[...] = p0_ref[...] * y01_ref[0] + p1_ref[...] * y01_ref[1]


def _combine(y01, p0, p1, interpret=False):
    return pl.pallas_call(
        _combine_body,
        grid=(1,),
        in_specs=[
            pl.BlockSpec((2, S, D), lambda i: (0, 0, 0)),
            pl.BlockSpec((S, 1), lambda i: (0, 0)),
            pl.BlockSpec((S, 1), lambda i: (0, 0)),
        ],
        out_specs=pl.BlockSpec((S, D), lambda i: (0, 0)),
        out_shape=jax.ShapeDtypeStruct((S, D), jnp.float32),
        interpret=interpret,
    )(y01.reshape(2, S, D), p0, p1)


def _run(xs, noisy, W1, b1, W2, b2, interpret=False,
         scatter=_scatter_x, gather=_gather_y):
    p0, p1, pos0, pos1, texp, runid, rexp = _router(noisy, interpret=interpret)
    pos01 = jnp.concatenate([pos0.reshape(-1), pos1.reshape(-1)])
    xg = scatter(xs, pos01)
    hg = _fc1(texp, runid, rexp, xg, W1, b1, interpret=interpret)
    yg = _fc2(texp, runid, rexp, hg, W2, b2, interpret=interpret)
    y01 = gather(yg, pos01)
    return _combine(y01, p0, p1, interpret=interpret)


def kernel(x, Wg, bg, Wn, bn, W1, b1, W2, b2, noise):
    logits = x @ Wg + bg
    noise_logits = x @ Wn + bn
    noisy = logits + noise * jax.nn.softplus(noise_logits)
    out = _run(x[0], noisy[0], W1, b1, W2, b2)
    return out[None]


# speedup vs baseline: 3.6993x; 1.0894x over previous
import functools
import math

import jax
import jax.numpy as jnp
from jax import lax
from jax.experimental import pallas as pl
from jax.experimental.pallas import tpu as pltpu
from jax.experimental.pallas import tpu_sc as plsc

S, D, E, K, H = 2048, 1024, 8, 2, 4096
HC = 512
NH = H // HC
TM = 256
NPAIR = K * S
MAX_SLOTS = 6144
MAX_TILES = MAX_SLOTS // TM
D2 = D // 2
NC, NS = 2, 16
NW = NC * NS
CHUNK = 64
NCHUNK = NPAIR // (NW * CHUNK)

_HIGHEST = jax.lax.Precision.HIGHEST


def _gelu(v):
    return 0.5 * v * (1.0 + jax.lax.erf(v * (1.0 / math.sqrt(2.0))))


NRUN = 16


def _router_body(noisy_ref, p0_ref, p1_ref, pos0_ref, pos1_ref, texp_ref,
                 runid_ref, rexp_ref):
    noisy = noisy_ref[...]
    lanes = jax.lax.broadcasted_iota(jnp.int32, (S, E), 1)

    v0 = jnp.max(noisy, axis=1, keepdims=True)
    e0 = jnp.min(jnp.where(noisy == v0, lanes, E), axis=1, keepdims=True)
    oh0 = lanes == e0
    masked = jnp.where(oh0, -jnp.inf, noisy)
    v1 = jnp.max(masked, axis=1, keepdims=True)
    e1 = jnp.min(jnp.where(masked == v1, lanes, E), axis=1, keepdims=True)
    oh1 = lanes == e1

    z = jnp.exp(v1 - v0)
    p0_ref[...] = 1.0 / (1.0 + z)
    p1_ref[...] = z / (1.0 + z)

    m = jnp.where(oh0 | oh1, 1.0, 0.0)
    r = jax.lax.broadcasted_iota(jnp.int32, (S, S), 0)
    c = jax.lax.broadcasted_iota(jnp.int32, (S, S), 1)
    tri = jnp.where(r > c, 1.0, 0.0)
    csum = jnp.dot(tri, m, preferred_element_type=jnp.float32)

    counts = jnp.sum(m, axis=0, keepdims=True)
    ntf = jnp.floor((counts + (TM - 1)) * (1.0 / TM))
    re = jax.lax.broadcasted_iota(jnp.int32, (E, E), 0)
    ce = jax.lax.broadcasted_iota(jnp.int32, (E, E), 1)
    trie = jnp.where(re < ce, 1.0, 0.0)
    btile = jnp.dot(ntf, trie,
                    preferred_element_type=jnp.float32)
    bsf = btile * float(TM)

    rank0 = jnp.sum(jnp.where(oh0, csum, 0.0), axis=1, keepdims=True)
    rank1 = jnp.sum(jnp.where(oh1, csum, 0.0), axis=1, keepdims=True)
    base0 = jnp.sum(jnp.where(oh0, bsf, 0.0), axis=1, keepdims=True)
    base1 = jnp.sum(jnp.where(oh1, bsf, 0.0), axis=1, keepdims=True)
    pos0_ref[...] = (base0 + rank0).astype(jnp.int32)
    pos1_ref[...] = (base1 + rank1).astype(jnp.int32)

    bt_col = jnp.reshape(btile, (E, 1))
    ti = jax.lax.broadcasted_iota(jnp.int32, (E, MAX_TILES), 1)
    texp = jnp.sum(jnp.where(bt_col <= ti.astype(jnp.float32), 1.0, 0.0),
                   axis=0, keepdims=True) - 1.0
    texp_ref[...] = texp.astype(jnp.int32)

    present_col = jnp.reshape(jnp.where(counts > 0.0, 1.0, 0.0), (E, 1))
    runidx_row = jnp.dot(jnp.reshape(present_col, (1, E)), trie,
                         preferred_element_type=jnp.float32)
    runidx_col = jnp.reshape(runidx_row, (E, 1))
    eid = jax.lax.broadcasted_iota(jnp.int32, (E, MAX_TILES), 0)
    sel = eid.astype(jnp.float32) == texp
    runid = jnp.sum(jnp.where(sel, runidx_col, 0.0), axis=0, keepdims=True)
    runid_ref[...] = runid.astype(jnp.int32)
    rr = jax.lax.broadcasted_iota(jnp.int32, (E, NRUN), 1).astype(jnp.float32)
    selr = jnp.logical_and(runidx_col == rr, present_col > 0.0)
    eidr = jax.lax.broadcasted_iota(jnp.int32, (E, NRUN), 0).astype(jnp.float32)
    rexp = jnp.sum(jnp.where(selr, eidr, 0.0), axis=0, keepdims=True)
    rexp_ref[...] = rexp.astype(jnp.int32)


def _router(noisy, interpret=False):
    return pl.pallas_call(
        _router_body,
        out_shape=(
            jax.ShapeDtypeStruct((S, 1), jnp.float32),
            jax.ShapeDtypeStruct((S, 1), jnp.float32),
            jax.ShapeDtypeStruct((S, 1), jnp.int32),
            jax.ShapeDtypeStruct((S, 1), jnp.int32),
            jax.ShapeDtypeStruct((1, MAX_TILES), jnp.int32),
            jax.ShapeDtypeStruct((1, MAX_TILES), jnp.int32),
            jax.ShapeDtypeStruct((1, NRUN), jnp.int32),
        ),
        interpret=interpret,
    )(noisy)


def _sc_mesh():
    return plsc.VectorSubcoreMesh(core_axis_name="c", subcore_axis_name="s")


def _scatter_x(xs, pos01):
    @functools.partial(
        pl.kernel,
        out_type=jax.ShapeDtypeStruct((MAX_SLOTS, D), jnp.float32),
        mesh=_sc_mesh(),
        scratch_types=[
            pltpu.VMEM((CHUNK,), jnp.int32),
            pltpu.VMEM((CHUNK, D), jnp.float32),
            pltpu.SemaphoreType.DMA,
        ],
    )
    def k(x_hbm, pos_hbm, xg_hbm, idx_v, rows_v, sem):
        wid = lax.axis_index("s") * NC + lax.axis_index("c")
        for c in range(NCHUNK):
            base = wid * (NCHUNK * CHUNK) + c * CHUNK
            src = lax.rem(base, S)
            pltpu.sync_copy(pos_hbm.at[pl.ds(base, CHUNK)], idx_v)
            pltpu.sync_copy(x_hbm.at[pl.ds(src, CHUNK)], rows_v)
            pltpu.async_copy(rows_v, xg_hbm.at[idx_v], sem).wait()

    return k(xs, pos01)


def _gather_y(yg, pos01):
    @functools.partial(
        pl.kernel,
        out_type=jax.ShapeDtypeStruct((NPAIR, D), jnp.float32),
        mesh=_sc_mesh(),
        scratch_types=[
            pltpu.VMEM((CHUNK,), jnp.int32),
            pltpu.VMEM((CHUNK, D), jnp.float32),
            pltpu.SemaphoreType.DMA,
        ],
    )
    def k(yg_hbm, pos_hbm, out_hbm, idx_v, rows_v, sem):
        wid = lax.axis_index("s") * NC + lax.axis_index("c")
        for c in range(NCHUNK):
            base = wid * (NCHUNK * CHUNK) + c * CHUNK
            pltpu.sync_copy(pos_hbm.at[pl.ds(base, CHUNK)], idx_v)
            pltpu.async_copy(yg_hbm.at[idx_v], rows_v, sem).wait()
            pltpu.sync_copy(rows_v, out_hbm.at[pl.ds(base, CHUNK)])

    return k(yg, pos01)


def _weight_prefetch(i, runid_ref, rexp_ref, w_hbm, wbuf, sems):
    rid = runid_ref[0, i]
    prev = runid_ref[0, jnp.maximum(i - 1, 0)]
    is_first = jnp.logical_or(i == 0, rid != prev)
    slot = jax.lax.rem(rid, 2)

    @pl.when(i == 0)
    def _():
        pltpu.make_async_copy(w_hbm.at[rexp_ref[0, 0]], wbuf.at[0],
                              sems.at[0]).start()
        pltpu.make_async_copy(w_hbm.at[rexp_ref[0, 1]], wbuf.at[1],
                              sems.at[1]).start()

    @pl.when(jnp.logical_and(is_first, rid >= 1))
    def _():
        pltpu.make_async_copy(w_hbm.at[rexp_ref[0, rid + 1]],
                              wbuf.at[1 - slot], sems.at[1 - slot]).start()

    @pl.when(is_first)
    def _():
        pltpu.make_async_copy(w_hbm.at[rexp_ref[0, rid]], wbuf.at[slot],
                              sems.at[slot]).wait()

    @pl.when(i == MAX_TILES - 1)
    def _():
        pltpu.make_async_copy(w_hbm.at[rexp_ref[0, rid + 1]],
                              wbuf.at[1 - slot], sems.at[1 - slot]).wait()

    return slot


def _fc1_body(texp_ref, runid_ref, rexp_ref, xg_ref, b1_ref, w1_hbm, hg_ref,
              wbuf, sems):
    i = pl.program_id(0)
    slot = _weight_prefetch(i, runid_ref, rexp_ref, w1_hbm, wbuf, sems)
    h = jnp.dot(xg_ref[...], wbuf[slot], preferred_element_type=jnp.float32)
    hg_ref[...] = _gelu(h + b1_ref[0]).astype(jnp.bfloat16)


def _fc1(texp, runid, rexp, xg, W1, b1, interpret=False):
    return pl.pallas_call(
        _fc1_body,
        grid_spec=pltpu.PrefetchScalarGridSpec(
            num_scalar_prefetch=3,
            grid=(MAX_TILES,),
            in_specs=[
                pl.BlockSpec((TM, D), lambda i, texp, runid, rexp: (i, 0)),
                pl.BlockSpec((1, 1, H),
                             lambda i, texp, runid, rexp: (texp[0, i], 0, 0)),
                pl.BlockSpec(memory_space=pl.ANY),
            ],
            out_specs=pl.BlockSpec((TM, H), lambda i, texp, runid, rexp: (i, 0)),
            scratch_shapes=[
                pltpu.VMEM((2, D, H), jnp.float32),
                pltpu.SemaphoreType.DMA((2,)),
            ],
        ),
        out_shape=jax.ShapeDtypeStruct((MAX_SLOTS, H), jnp.bfloat16),
        compiler_params=pltpu.CompilerParams(
            dimension_semantics=("arbitrary",),
        ),
        interpret=interpret,
    )(texp, runid, rexp, xg, b1.reshape(E, 1, H), W1)


def _fc2_body(texp_ref, runid_ref, rexp_ref, hg_ref, b2_ref, w2_hbm, yg_ref,
              wbuf, sems):
    i = pl.program_id(0)
    slot = _weight_prefetch(i, runid_ref, rexp_ref, w2_hbm, wbuf, sems)
    y = jnp.dot(hg_ref[...].astype(jnp.float32), wbuf[slot],
                preferred_element_type=jnp.float32)
    yg_ref[...] = y + b2_ref[0]


def _fc2(texp, runid, rexp, hg, W2, b2, interpret=False):
    return pl.pallas_call(
        _fc2_body,
        grid_spec=pltpu.PrefetchScalarGridSpec(
            num_scalar_prefetch=3,
            grid=(MAX_TILES,),
            in_specs=[
                pl.BlockSpec((TM, H), lambda i, texp, runid, rexp: (i, 0)),
                pl.BlockSpec((1, 1, D),
                             lambda i, texp, runid, rexp: (texp[0, i], 0, 0)),
                pl.BlockSpec(memory_space=pl.ANY),
            ],
            out_specs=pl.BlockSpec((TM, D), lambda i, texp, runid, rexp: (i, 0)),
            scratch_shapes=[
                pltpu.VMEM((2, H, D), jnp.float32),
                pltpu.SemaphoreType.DMA((2,)),
            ],
        ),
        out_shape=jax.ShapeDtypeStruct((MAX_SLOTS, D), jnp.float32),
        compiler_params=pltpu.CompilerParams(
            dimension_semantics=("arbitrary",),
        ),
        interpret=interpret,
    )(texp, runid, rexp, hg, b2.reshape(E, 1, D), W2)


def _combine_body(y01_ref, p0_ref, p1_ref, out_ref):
    out_ref[...] = p0_ref[...] * y01_ref[0] + p1_ref[...] * y01_ref[1]


def _combine(y01, p0, p1, interpret=False):
    return pl.pallas_call(
        _combine_body,
        grid=(1,),
        in_specs=[
            pl.BlockSpec((2, S, D), lambda i: (0, 0, 0)),
            pl.BlockSpec((S, 1), lambda i: (0, 0)),
            pl.BlockSpec((S, 1), lambda i: (0, 0)),
        ],
        out_specs=pl.BlockSpec((S, D), lambda i: (0, 0)),
        out_shape=jax.ShapeDtypeStruct((S, D), jnp.float32),
        interpret=interpret,
    )(y01.reshape(2, S, D), p0, p1)


def _run(xs, noisy, W1, b1, W2, b2, interpret=False,
         scatter=_scatter_x, gather=_gather_y):
    p0, p1, pos0, pos1, texp, runid, rexp = _router(noisy, interpret=interpret)
    pos01 = jnp.concatenate([pos0.reshape(-1), pos1.reshape(-1)])
    xg = scatter(xs, pos01)
    hg = _fc1(texp, runid, rexp, xg, W1, b1, interpret=interpret)
    yg = _fc2(texp, runid, rexp, hg, W2, b2, interpret=interpret)
    y01 = gather(yg, pos01)
    return _combine(y01, p0, p1, interpret=interpret)


def kernel(x, Wg, bg, Wn, bn, W1, b1, W2, b2, noise):
    logits = x @ Wg + bg
    noise_logits = x @ Wn + bn
    noisy = logits + noise * jax.nn.softplus(noise_logits)
    out = _run(x[0], noisy[0], W1, b1, W2, b2)
    return out[None]


# speedup vs baseline: 4.0503x; 1.0949x over previous
import functools
import math

import jax
import jax.numpy as jnp
from jax import lax
from jax.experimental import pallas as pl
from jax.experimental.pallas import tpu as pltpu
from jax.experimental.pallas import tpu_sc as plsc

S, D, E, K, H = 2048, 1024, 8, 2, 4096
HC = 512
NH = H // HC
TM = 256
NPAIR = K * S
MAX_SLOTS = 6144
MAX_TILES = MAX_SLOTS // TM
D2 = D // 2
NC, NS = 2, 16
NW = NC * NS
CHUNK = 64
NCHUNK = NPAIR // (NW * CHUNK)

_HIGHEST = jax.lax.Precision.HIGHEST


def _gelu(v):
    return 0.5 * v * (1.0 + jax.lax.erf(v * (1.0 / math.sqrt(2.0))))


NRUN = 16


def _router_body(noisy_ref, p0_ref, p1_ref, pos0_ref, pos1_ref, texp_ref,
                 runid_ref, rexp_ref, padf_ref):
    noisy = noisy_ref[...]
    lanes = jax.lax.broadcasted_iota(jnp.int32, (S, E), 1)

    v0 = jnp.max(noisy, axis=1, keepdims=True)
    e0 = jnp.min(jnp.where(noisy == v0, lanes, E), axis=1, keepdims=True)
    oh0 = lanes == e0
    masked = jnp.where(oh0, -jnp.inf, noisy)
    v1 = jnp.max(masked, axis=1, keepdims=True)
    e1 = jnp.min(jnp.where(masked == v1, lanes, E), axis=1, keepdims=True)
    oh1 = lanes == e1

    z = jnp.exp(v1 - v0)
    p0_ref[...] = 1.0 / (1.0 + z)
    p1_ref[...] = z / (1.0 + z)

    m = jnp.where(oh0 | oh1, 1.0, 0.0)
    r = jax.lax.broadcasted_iota(jnp.int32, (S, S), 0)
    c = jax.lax.broadcasted_iota(jnp.int32, (S, S), 1)
    tri = jnp.where(r > c, 1.0, 0.0)
    csum = jnp.dot(tri, m, preferred_element_type=jnp.float32)

    counts = jnp.sum(m, axis=0, keepdims=True)
    ntf = jnp.floor((counts + (TM - 1)) * (1.0 / TM))
    re = jax.lax.broadcasted_iota(jnp.int32, (E, E), 0)
    ce = jax.lax.broadcasted_iota(jnp.int32, (E, E), 1)
    trie = jnp.where(re < ce, 1.0, 0.0)
    btile = jnp.dot(ntf, trie,
                    preferred_element_type=jnp.float32)
    bsf = btile * float(TM)

    rank0 = jnp.sum(jnp.where(oh0, csum, 0.0), axis=1, keepdims=True)
    rank1 = jnp.sum(jnp.where(oh1, csum, 0.0), axis=1, keepdims=True)
    base0 = jnp.sum(jnp.where(oh0, bsf, 0.0), axis=1, keepdims=True)
    base1 = jnp.sum(jnp.where(oh1, bsf, 0.0), axis=1, keepdims=True)
    pos0_ref[...] = (base0 + rank0).astype(jnp.int32)
    pos1_ref[...] = (base1 + rank1).astype(jnp.int32)

    bt_col = jnp.reshape(btile, (E, 1))
    ti = jax.lax.broadcasted_iota(jnp.int32, (E, MAX_TILES), 1)
    texp = jnp.sum(jnp.where(bt_col <= ti.astype(jnp.float32), 1.0, 0.0),
                   axis=0, keepdims=True) - 1.0
    texp_ref[...] = texp.astype(jnp.int32)

    present_col = jnp.reshape(jnp.where(counts > 0.0, 1.0, 0.0), (E, 1))
    runidx_row = jnp.dot(jnp.reshape(present_col, (1, E)), trie,
                         preferred_element_type=jnp.float32)
    runidx_col = jnp.reshape(runidx_row, (E, 1))
    eid = jax.lax.broadcasted_iota(jnp.int32, (E, MAX_TILES), 0)
    sel = eid.astype(jnp.float32) == texp
    runid = jnp.sum(jnp.where(sel, runidx_col, 0.0), axis=0, keepdims=True)
    runid_ref[...] = runid.astype(jnp.int32)
    rr = jax.lax.broadcasted_iota(jnp.int32, (E, NRUN), 1).astype(jnp.float32)
    selr = jnp.logical_and(runidx_col == rr, present_col > 0.0)
    eidr = jax.lax.broadcasted_iota(jnp.int32, (E, NRUN), 0).astype(jnp.float32)
    rexp = jnp.sum(jnp.where(selr, eidr, 0.0), axis=0, keepdims=True)
    rexp_ref[...] = rexp.astype(jnp.int32)

    total = jnp.sum(ntf, axis=1, keepdims=True)
    ti1 = jax.lax.broadcasted_iota(jnp.int32, (1, MAX_TILES), 1)
    padf_ref[...] = jnp.where(ti1.astype(jnp.float32) >= total, 1, 0)


def _router(noisy, interpret=False):
    return pl.pallas_call(
        _router_body,
        out_shape=(
            jax.ShapeDtypeStruct((S, 1), jnp.float32),
            jax.ShapeDtypeStruct((S, 1), jnp.float32),
            jax.ShapeDtypeStruct((S, 1), jnp.int32),
            jax.ShapeDtypeStruct((S, 1), jnp.int32),
            jax.ShapeDtypeStruct((1, MAX_TILES), jnp.int32),
            jax.ShapeDtypeStruct((1, MAX_TILES), jnp.int32),
            jax.ShapeDtypeStruct((1, NRUN), jnp.int32),
            jax.ShapeDtypeStruct((1, MAX_TILES), jnp.int32),
        ),
        interpret=interpret,
    )(noisy)


def _sc_mesh():
    return plsc.VectorSubcoreMesh(core_axis_name="c", subcore_axis_name="s")


def _scatter_x(xs, pos01):
    @functools.partial(
        pl.kernel,
        out_type=jax.ShapeDtypeStruct((MAX_SLOTS, D), jnp.float32),
        mesh=_sc_mesh(),
        scratch_types=[
            pltpu.VMEM((CHUNK,), jnp.int32),
            pltpu.VMEM((CHUNK, D), jnp.float32),
            pltpu.SemaphoreType.DMA,
        ],
    )
    def k(x_hbm, pos_hbm, xg_hbm, idx_v, rows_v, sem):
        wid = lax.axis_index("s") * NC + lax.axis_index("c")
        for c in range(NCHUNK):
            base = wid * (NCHUNK * CHUNK) + c * CHUNK
            src = lax.rem(base, S)
            pltpu.sync_copy(pos_hbm.at[pl.ds(base, CHUNK)], idx_v)
            pltpu.sync_copy(x_hbm.at[pl.ds(src, CHUNK)], rows_v)
            pltpu.async_copy(rows_v, xg_hbm.at[idx_v], sem).wait()

    return k(xs, pos01)


def _gather_y(yg, pos01):
    @functools.partial(
        pl.kernel,
        out_type=jax.ShapeDtypeStruct((NPAIR, D), jnp.float32),
        mesh=_sc_mesh(),
        scratch_types=[
            pltpu.VMEM((CHUNK,), jnp.int32),
            pltpu.VMEM((CHUNK, D), jnp.float32),
            pltpu.SemaphoreType.DMA,
        ],
    )
    def k(yg_hbm, pos_hbm, out_hbm, idx_v, rows_v, sem):
        wid = lax.axis_index("s") * NC + lax.axis_index("c")
        for c in range(NCHUNK):
            base = wid * (NCHUNK * CHUNK) + c * CHUNK
            pltpu.sync_copy(pos_hbm.at[pl.ds(base, CHUNK)], idx_v)
            pltpu.async_copy(yg_hbm.at[idx_v], rows_v, sem).wait()
            pltpu.sync_copy(rows_v, out_hbm.at[pl.ds(base, CHUNK)])

    return k(yg, pos01)


def _weight_prefetch(i, runid_ref, rexp_ref, w_hbm, wbuf, sems):
    rid = runid_ref[0, i]
    prev = runid_ref[0, jnp.maximum(i - 1, 0)]
    is_first = jnp.logical_or(i == 0, rid != prev)
    slot = jax.lax.rem(rid, 2)

    @pl.when(i == 0)
    def _():
        pltpu.make_async_copy(w_hbm.at[rexp_ref[0, 0]], wbuf.at[0],
                              sems.at[0]).start()
        pltpu.make_async_copy(w_hbm.at[rexp_ref[0, 1]], wbuf.at[1],
                              sems.at[1]).start()

    @pl.when(jnp.logical_and(is_first, rid >= 1))
    def _():
        pltpu.make_async_copy(w_hbm.at[rexp_ref[0, rid + 1]],
                              wbuf.at[1 - slot], sems.at[1 - slot]).start()

    @pl.when(is_first)
    def _():
        pltpu.make_async_copy(w_hbm.at[rexp_ref[0, rid]], wbuf.at[slot],
                              sems.at[slot]).wait()

    @pl.when(i == MAX_TILES - 1)
    def _():
        pltpu.make_async_copy(w_hbm.at[rexp_ref[0, rid + 1]],
                              wbuf.at[1 - slot], sems.at[1 - slot]).wait()

    return slot


def _tile_idx(i, padf):
    return jnp.where(padf[0, i] == 1, MAX_TILES - 1, i)


def _fc1_body(texp_ref, runid_ref, rexp_ref, padf_ref, xg_ref, b1_ref,
              w1_hbm, hg_ref, wbuf, sems):
    i = pl.program_id(0)
    slot = _weight_prefetch(i, runid_ref, rexp_ref, w1_hbm, wbuf, sems)

    @pl.when(padf_ref[0, i] == 0)
    def _():
        h = jnp.dot(xg_ref[...], wbuf[slot],
                    preferred_element_type=jnp.float32)
        hg_ref[...] = _gelu(h + b1_ref[0]).astype(jnp.bfloat16)


def _fc1(texp, runid, rexp, padf, xg, W1, b1, interpret=False):
    return pl.pallas_call(
        _fc1_body,
        grid_spec=pltpu.PrefetchScalarGridSpec(
            num_scalar_prefetch=4,
            grid=(MAX_TILES,),
            in_specs=[
                pl.BlockSpec((TM, D),
                             lambda i, texp, runid, rexp, padf:
                             (_tile_idx(i, padf), 0)),
                pl.BlockSpec((1, 1, H),
                             lambda i, texp, runid, rexp, padf:
                             (texp[0, i], 0, 0)),
                pl.BlockSpec(memory_space=pl.ANY),
            ],
            out_specs=pl.BlockSpec((TM, H),
                                   lambda i, texp, runid, rexp, padf:
                                   (_tile_idx(i, padf), 0)),
            scratch_shapes=[
                pltpu.VMEM((2, D, H), jnp.float32),
                pltpu.SemaphoreType.DMA((2,)),
            ],
        ),
        out_shape=jax.ShapeDtypeStruct((MAX_SLOTS, H), jnp.bfloat16),
        compiler_params=pltpu.CompilerParams(
            dimension_semantics=("arbitrary",),
        ),
        interpret=interpret,
    )(texp, runid, rexp, padf, xg, b1.reshape(E, 1, H), W1)


def _fc2_body(texp_ref, runid_ref, rexp_ref, padf_ref, hg_ref, b2_ref,
              w2_hbm, yg_ref, wbuf, sems):
    i = pl.program_id(0)
    slot = _weight_prefetch(i, runid_ref, rexp_ref, w2_hbm, wbuf, sems)

    @pl.when(padf_ref[0, i] == 0)
    def _():
        y = jnp.dot(hg_ref[...].astype(jnp.float32), wbuf[slot],
                    preferred_element_type=jnp.float32)
        yg_ref[...] = y + b2_ref[0]


def _fc2(texp, runid, rexp, padf, hg, W2, b2, interpret=False):
    return pl.pallas_call(
        _fc2_body,
        grid_spec=pltpu.PrefetchScalarGridSpec(
            num_scalar_prefetch=4,
            grid=(MAX_TILES,),
            in_specs=[
                pl.BlockSpec((TM, H),
                             lambda i, texp, runid, rexp, padf:
                             (_tile_idx(i, padf), 0)),
                pl.BlockSpec((1, 1, D),
                             lambda i, texp, runid, rexp, padf:
                             (texp[0, i], 0, 0)),
                pl.BlockSpec(memory_space=pl.ANY),
            ],
            out_specs=pl.BlockSpec((TM, D),
                                   lambda i, texp, runid, rexp, padf:
                                   (_tile_idx(i, padf), 0)),
            scratch_shapes=[
                pltpu.VMEM((2, H, D), jnp.float32),
                pltpu.SemaphoreType.DMA((2,)),
            ],
        ),
        out_shape=jax.ShapeDtypeStruct((MAX_SLOTS, D), jnp.float32),
        compiler_params=pltpu.CompilerParams(
            dimension_semantics=("arbitrary",),
        ),
        interpret=interpret,
    )(texp, runid, rexp, padf, hg, b2.reshape(E, 1, D), W2)


def _combine_body(y01_ref, p0_ref, p1_ref, out_ref):
    out_ref[...] = p0_ref[...] * y01_ref[0] + p1_ref[...] * y01_ref[1]


def _combine(y01, p0, p1, interpret=False):
    return pl.pallas_call(
        _combine_body,
        grid=(1,),
        in_specs=[
            pl.BlockSpec((2, S, D), lambda i: (0, 0, 0)),
            pl.BlockSpec((S, 1), lambda i: (0, 0)),
            pl.BlockSpec((S, 1), lambda i: (0, 0)),
        ],
        out_specs=pl.BlockSpec((S, D), lambda i: (0, 0)),
        out_shape=jax.ShapeDtypeStruct((S, D), jnp.float32),
        interpret=interpret,
    )(y01.reshape(2, S, D), p0, p1)


def _run(xs, noisy, W1, b1, W2, b2, interpret=False,
         scatter=_scatter_x, gather=_gather_y):
    (p0, p1, pos0, pos1, texp, runid, rexp, padf) = _router(
        noisy, interpret=interpret)
    pos01 = jnp.concatenate([pos0.reshape(-1), pos1.reshape(-1)])
    xg = scatter(xs, pos01)
    hg = _fc1(texp, runid, rexp, padf, xg, W1, b1, interpret=interpret)
    yg = _fc2(texp, runid, rexp, padf, hg, W2, b2, interpret=interpret)
    y01 = gather(yg, pos01)
    return _combine(y01, p0, p1, interpret=interpret)


def kernel(x, Wg, bg, Wn, bn, W1, b1, W2, b2, noise):
    logits = x @ Wg + bg
    noise_logits = x @ Wn + bn
    noisy = logits + noise * jax.nn.softplus(noise_logits)
    out = _run(x[0], noisy[0], W1, b1, W2, b2)
    return out[None]
